# Initial kernel scaffold; baseline (speedup 1.0000x reference)
import jax
import jax.numpy as jnp
from jax.experimental import pallas as pl


def kernel(obs, W0, W1, W2, W3, W4, W5, W6, W7, W8, W9):
    raise NotImplementedError("write your pallas kernel here")


# speedup vs baseline: 4.2092x; 4.2092x over previous
import functools

import jax
import jax.numpy as jnp
from jax.experimental import pallas as pl
from jax.experimental.pallas import tpu as pltpu

B = 1024
N = 100
NP = 104
C = 32
C2 = 64
KNN = 10
EPS = 1e-5
NEG = -1e30

_INTERPRET = False


def _dot(x, w, dims):
  return jax.lax.dot_general(x, w, dims, preferred_element_type=jnp.float32)


def _leaky(x):
  return jnp.where(x > 0, x, 0.2 * x)


def _bnin(h, mi):
  ch = mi.shape[-1]
  tgt = (1,) * (h.ndim - 1) + (ch,)
  m = mi[0:1, :].reshape(tgt)
  s = mi[1:2, :].reshape(tgt)
  return _leaky((h - m) / s)


def _rowmm(x3, w):
  b, n, c = x3.shape
  y = _dot(x3.reshape(b * n, c), w, (((1,), (0,)), ((), ())))
  return y.reshape(b, n, w.shape[1])


def _rowsum(x3, rmask):
  b, n, c = x3.shape
  xm = jnp.where(rmask, x3, 0.0).reshape(b * n, c)
  return jnp.sum(xm, axis=0, keepdims=True)


def _k0_body(obs_ref, w_ref, h_ref, s_ref, q_ref):
  x = obs_ref[...]
  h = _dot(x, w_ref[...], (((1,), (0,)), ((), ())))
  h_ref[...] = h
  s_ref[...] = jnp.sum(h, axis=0, keepdims=True)[None]
  q_ref[...] = jnp.sum(h * h, axis=0, keepdims=True)[None]


def _pairdist(x, xx, bb):
  g = _dot(x, x, (((2,), (2,)), ((0,), (0,))))
  inner = -2.0 * g
  xx_l = jnp.swapaxes(xx, 1, 2)
  pd = (-xx_l) - inner
  pd = pd - xx
  jj = jax.lax.broadcasted_iota(jnp.int32, (bb, NP, NP), 2)
  return jnp.where(jj < N, pd, NEG), jj


def _gather_all(a, amf, bb, ch):
  a_t = jnp.swapaxes(a, 1, 2)
  am_t = jnp.swapaxes(amf, 1, 2)
  parts = []
  for k in range(KNN):
    ik = am_t[:, k:k + 1, :].astype(jnp.int32)
    ikb = jnp.broadcast_to(ik, (bb, ch, NP))
    parts.append(
        jnp.take_along_axis(a_t, ikb, axis=2, mode="promise_in_bounds"))
  gall_t = jnp.concatenate(parts, axis=2)
  g = jnp.swapaxes(gall_t, 1, 2)
  return g.reshape(bb, KNN, NP, ch)


def _edge_feature(x, amf, bb):
  gx = _gather_all(x, amf, bb, C)
  xb = jnp.broadcast_to(x[:, None, :, :], (bb, KNN, NP, C))
  return jnp.concatenate([gx - xb, xb], axis=3)


def _edge_mm(f4, w, bb):
  co = w.shape[1]
  y = _dot(f4.reshape(bb * KNN * NP, f4.shape[3]), w, (((1,), (0,)), ((), ())))
  return y.reshape(bb, KNN, NP, co)


def _edge_sums(h4, bb):
  rmask = jax.lax.broadcasted_iota(jnp.int32, (bb, KNN, NP, 1), 2) < N
  hm = jnp.where(rmask, h4, 0.0).reshape(bb * KNN * NP, C)
  hq = jnp.where(rmask, h4 * h4, 0.0).reshape(bb * KNN * NP, C)
  return (jnp.sum(hm, axis=0, keepdims=True)[None],
          jnp.sum(hq, axis=0, keepdims=True)[None])


def _topk(pd, jj, o_ref, store_idx):
  amfs = []
  for k in range(KNN):
    rmax = jnp.max(pd, axis=2, keepdims=True)
    am = jnp.min(jnp.where(pd == rmax, jj, NP), axis=2, keepdims=True)
    pd = jnp.where(jj == am, NEG, pd)
    if store_idx:
      o_ref[:, :, k:k + 1] = am
    amfs.append(am.astype(jnp.float32))
  return jnp.concatenate(amfs, axis=2)


def _knn_body(x_ref, xx_ref, w1_ref, o_ref, s_ref, *rest, bb, store_idx):
  q_ref = rest[0] if rest else None
  rmask = jax.lax.broadcasted_iota(jnp.int32, (bb, NP, 1), 1) < N
  x = jnp.where(rmask, x_ref[...], 0.0)
  pd, jj = _pairdist(x, xx_ref[...], bb)
  amf = _topk(pd, jj, o_ref, store_idx)
  f4 = _edge_feature(x, amf, bb)
  h1 = _edge_mm(f4, w1_ref[...], bb)
  if store_idx:
    s_ref[...] = h1
  else:
    sv, qv = _edge_sums(h1, bb)
    s_ref[...] = sv
    q_ref[...] = qv
    o_ref[...] = jnp.max(h1, axis=1)


def _apply_body(x_ref, idx_ref, mih_ref, w1_ref, w2_ref,
                m_ref, s_ref, *, bb):
  rmask = jax.lax.broadcasted_iota(jnp.int32, (bb, NP, 1), 1) < N
  x = jnp.where(rmask, x_ref[...], 0.0)
  f4 = _edge_feature(x, idx_ref[...].astype(jnp.float32), bb)
  h1 = _edge_mm(f4, w1_ref[...], bb)
  e = _bnin(h1, mih_ref[...])
  h2 = _edge_mm(e, w2_ref[...], bb)
  s_ref[...] = h2
  m_ref[...] = jnp.max(h2, axis=1)


def _tail_a_body(m2_ref, m4_ref, m5_ref, mi2_ref, mi4_ref, mi5_ref, w6_ref,
                 g_ref, s_ref, q_ref, *, bt):
  x1 = _bnin(m2_ref[...], mi2_ref[...])
  x2 = _bnin(m4_ref[...], mi4_ref[...])
  x3 = _bnin(m5_ref[...], mi5_ref[...])
  cat = jnp.concatenate([x1, x2, x3], axis=2)
  h6 = _rowmm(cat, w6_ref[...])
  rmask = jax.lax.broadcasted_iota(jnp.int32, (bt, NP, 1), 1) < N
  s_ref[...] = _rowsum(h6, rmask)[None]
  q_ref[...] = _rowsum(h6 * h6, rmask)[None]
  g_ref[...] = jnp.max(jnp.where(rmask, h6, NEG), axis=1)


def _tail_b_body(m2_ref, m4_ref, m5_ref, g6_ref, mi2_ref, mi4_ref, mi5_ref,
                 mi6_ref, w7_ref, h7_ref, s_ref, q_ref, *, bt):
  x1 = _bnin(m2_ref[...], mi2_ref[...])
  x2 = _bnin(m4_ref[...], mi4_ref[...])
  x3 = _bnin(m5_ref[...], mi5_ref[...])
  gx = _bnin(g6_ref[...], mi6_ref[...])
  gb = jnp.broadcast_to(gx[:, None, :], (bt, NP, gx.shape[-1]))
  cat = jnp.concatenate([gb, x1, x2, x3], axis=2)
  h7 = _rowmm(cat, w7_ref[...])
  rmask = jax.lax.broadcasted_iota(jnp.int32, (bt, NP, 1), 1) < N
  h7_ref[...] = h7
  s_ref[...] = _rowsum(h7, rmask)[None]
  q_ref[...] = _rowsum(h7 * h7, rmask)[None]


def _tail_c_body(h7_ref, mi7_ref, w8_ref, h8_ref, s_ref, q_ref, *, bt):
  e = _bnin(h7_ref[...], mi7_ref[...])
  h8 = _rowmm(e, w8_ref[...])
  rmask = jax.lax.broadcasted_iota(jnp.int32, (bt, NP, 1), 1) < N
  h8_ref[...] = h8
  s_ref[...] = _rowsum(h8, rmask)[None]
  q_ref[...] = _rowsum(h8 * h8, rmask)[None]


def _tail_d_body(h8_ref, mi8_ref, w9_ref, o_ref, *, bt):
  e = _bnin(h8_ref[...], mi8_ref[...])
  o_ref[...] = _rowmm(e, w9_ref[...])


def _full(shape):
  nd = len(shape)
  return pl.BlockSpec(shape, lambda i: (0,) * nd)


def _stats(s, q, cnt):
  st = jnp.sum(s.reshape(-1, s.shape[-1]), axis=0)
  qt = jnp.sum(q.reshape(-1, q.shape[-1]), axis=0)
  m = st / cnt
  v = qt / cnt - m * m
  return jnp.stack([m, jnp.sqrt(v + EPS)])


def _mi_exact(h4):
  hv = h4[:, :, :N, :]
  m = jnp.mean(hv, axis=(0, 1, 2))
  v = jnp.var(hv, axis=(0, 1, 2))
  return jnp.stack([m, jnp.sqrt(v + EPS)])


def _pc(body, grid, in_specs, out_shapes, out_specs):
  return pl.pallas_call(
      body, grid=grid, in_specs=in_specs, out_shape=out_shapes,
      out_specs=out_specs, interpret=_INTERPRET)


def kernel(obs, W0, W1, W2, W3, W4, W5, W6, W7, W8, W9):
  f32 = jnp.float32
  cnt_p = float(B * N)
  cnt_e = float(B * N * KNN)

  obs3 = jnp.pad(obs.reshape(B, N, 6), ((0, 0), (0, NP - N), (0, 0)))
  obs2 = obs3.reshape(B * NP, 6)

  w0t = W0.T
  w1t, w2t = W1.T, W2.T
  w3t, w4t = W3.T, W4.T
  w5t = W5.T
  w6t = W6.T
  w7t = W7.T
  w8t = W8.T
  w9t = W9.T

  bb0, g0 = 64, B // 64
  h0, s0, q0 = _pc(
      _k0_body, (g0,),
      [pl.BlockSpec((bb0 * NP, 6), lambda i: (i, 0)), _full((6, C))],
      [jax.ShapeDtypeStruct((B * NP, C), f32),
       jax.ShapeDtypeStruct((g0, 1, C), f32),
       jax.ShapeDtypeStruct((g0, 1, C), f32)],
      [pl.BlockSpec((bb0 * NP, C), lambda i: (i, 0)),
       pl.BlockSpec((1, 1, C), lambda i: (i, 0, 0)),
       pl.BlockSpec((1, 1, C), lambda i: (i, 0, 0))],
  )(obs2, w0t)
  h0 = h0.reshape(B, NP, C)
  del s0, q0

  def xbn(h3, mi):
    return _leaky((h3 - mi[0]) / mi[1])

  def xnorm(x3):
    xt = jnp.transpose(x3[:, :N, :], (0, 2, 1))
    xx = jnp.sum(xt ** 2, axis=1)
    return jnp.pad(xx, ((0, 0), (0, NP - N)))[:, :, None]

  def mi_point(h3):
    hv = h3[:, :N, :]
    return jnp.stack([jnp.mean(hv, axis=(0, 1)),
                      jnp.sqrt(jnp.var(hv, axis=(0, 1)) + EPS)])

  mi0 = mi_point(h0)
  x0 = xbn(h0, mi0)
  xx0 = xnorm(x0)

  bbk, gk = 16, B // 16
  bba, ga = 16, B // 16

  def knn_stage(x, xx, w1, store_idx):
    body = functools.partial(_knn_body, bb=bbk, store_idx=store_idx)
    oshape = (B, NP, KNN) if store_idx else (B, NP, C)
    odt = jnp.int32 if store_idx else f32
    ob = (bbk, NP, KNN) if store_idx else (bbk, NP, C)
    if store_idx:
      outs = [jax.ShapeDtypeStruct(oshape, odt),
              jax.ShapeDtypeStruct((B, KNN, NP, C), f32)]
      ospecs = [pl.BlockSpec(ob, lambda i: (i, 0, 0)),
                pl.BlockSpec((bbk, KNN, NP, C), lambda i: (i, 0, 0, 0))]
    else:
      outs = [jax.ShapeDtypeStruct(oshape, odt),
              jax.ShapeDtypeStruct((gk, 1, C), f32),
              jax.ShapeDtypeStruct((gk, 1, C), f32)]
      ospecs = [pl.BlockSpec(ob, lambda i: (i, 0, 0)),
                pl.BlockSpec((1, 1, C), lambda i: (i, 0, 0)),
                pl.BlockSpec((1, 1, C), lambda i: (i, 0, 0))]
    return _pc(
        body, (gk,),
        [pl.BlockSpec((bbk, NP, C), lambda i: (i, 0, 0)),
         pl.BlockSpec((bbk, NP, 1), lambda i: (i, 0, 0)),
         _full((C2, C))],
        outs, ospecs,
    )(x, xx, w1)

  def apply_stage(x, idx, mih, w1, w2):
    body = functools.partial(_apply_body, bb=bba)
    return _pc(
        body, (ga,),
        [pl.BlockSpec((bba, NP, C), lambda i: (i, 0, 0)),
         pl.BlockSpec((bba, NP, KNN), lambda i: (i, 0, 0)), _full((2, C)),
         _full((C2, C)), _full((C, C))],
        [jax.ShapeDtypeStruct((B, NP, C), f32),
         jax.ShapeDtypeStruct((B, KNN, NP, C), f32)],
        [pl.BlockSpec((bba, NP, C), lambda i: (i, 0, 0)),
         pl.BlockSpec((bba, KNN, NP, C), lambda i: (i, 0, 0, 0))],
    )(x, idx, mih, w1, w2)

  idx1, h1f = knn_stage(x0, xx0, w1t, True)
  mi1 = _mi_exact(h1f)
  m2, h2f = apply_stage(x0, idx1, mi1, w1t, w2t)
  mi2 = _mi_exact(h2f)
  x1 = xbn(m2, mi2)
  xx1 = xnorm(x1)

  idx2, h3f = knn_stage(x1, xx1, w3t, True)
  mi3 = _mi_exact(h3f)
  m4, h4f = apply_stage(x1, idx2, mi3, w3t, w4t)
  mi4 = _mi_exact(h4f)
  x2 = xbn(m4, mi4)
  xx2 = xnorm(x2)

  m5, s5, q5 = knn_stage(x2, xx2, w5t, False)
  mi5 = _stats(s5, q5, cnt_e)

  bt4, g4 = 32, B // 32
  g6, s6, q6 = _pc(
      functools.partial(_tail_a_body, bt=bt4), (g4,),
      [pl.BlockSpec((bt4, NP, C), lambda i: (i, 0, 0)),
       pl.BlockSpec((bt4, NP, C), lambda i: (i, 0, 0)),
       pl.BlockSpec((bt4, NP, C), lambda i: (i, 0, 0)),
       _full((2, C)), _full((2, C)), _full((2, C)), _full((3 * C, 512))],
      [jax.ShapeDtypeStruct((B, 512), f32),
       jax.ShapeDtypeStruct((g4, 1, 512), f32),
       jax.ShapeDtypeStruct((g4, 1, 512), f32)],
      [pl.BlockSpec((bt4, 512), lambda i: (i, 0)),
       pl.BlockSpec((1, 1, 512), lambda i: (i, 0, 0)),
       pl.BlockSpec((1, 1, 512), lambda i: (i, 0, 0))],
  )(m2, m4, m5, mi2, mi4, mi5, w6t)
  mi6 = _stats(s6, q6, cnt_p)

  bt5, g5 = 16, B // 16
  h7, s7, q7 = _pc(
      functools.partial(_tail_b_body, bt=bt5), (g5,),
      [pl.BlockSpec((bt5, NP, C), lambda i: (i, 0, 0)),
       pl.BlockSpec((bt5, NP, C), lambda i: (i, 0, 0)),
       pl.BlockSpec((bt5, NP, C), lambda i: (i, 0, 0)),
       pl.BlockSpec((bt5, 512), lambda i: (i, 0)),
       _full((2, C)), _full((2, C)), _full((2, C)), _full((2, 512)),
       _full((512 + 3 * C, 128))],
      [jax.ShapeDtypeStruct((B, NP, 128), f32),
       jax.ShapeDtypeStruct((g5, 1, 128), f32),
       jax.ShapeDtypeStruct((g5, 1, 128), f32)],
      [pl.BlockSpec((bt5, NP, 128), lambda i: (i, 0, 0)),
       pl.BlockSpec((1, 1, 128), lambda i: (i, 0, 0)),
       pl.BlockSpec((1, 1, 128), lambda i: (i, 0, 0))],
  )(m2, m4, m5, g6, mi2, mi4, mi5, mi6, w7t)
  mi7 = _stats(s7, q7, cnt_p)

  bt6, g6n = 64, B // 64
  h8, s8, q8 = _pc(
      functools.partial(_tail_c_body, bt=bt6), (g6n,),
      [pl.BlockSpec((bt6, NP, 128), lambda i: (i, 0, 0)), _full((2, 128)),
       _full((128, C))],
      [jax.ShapeDtypeStruct((B, NP, C), f32),
       jax.ShapeDtypeStruct((g6n, 1, C), f32),
       jax.ShapeDtypeStruct((g6n, 1, C), f32)],
      [pl.BlockSpec((bt6, NP, C), lambda i: (i, 0, 0)),
       pl.BlockSpec((1, 1, C), lambda i: (i, 0, 0)),
       pl.BlockSpec((1, 1, C), lambda i: (i, 0, 0))],
  )(h7, mi7, w8t)
  mi8 = _stats(s8, q8, cnt_p)

  bt7, g7 = 128, B // 128
  out = _pc(
      functools.partial(_tail_d_body, bt=bt7), (g7,),
      [pl.BlockSpec((bt7, NP, C), lambda i: (i, 0, 0)), _full((2, C)),
       _full((C, 1))],
      [jax.ShapeDtypeStruct((B, NP, 1), f32)],
      [pl.BlockSpec((bt7, NP, 1), lambda i: (i, 0, 0))],
  )(h8, mi8, w9t)[0]

  return out.reshape(B, NP)[:, :N]
